# Initial kernel scaffold; baseline (speedup 1.0000x reference)
import jax
import jax.numpy as jnp
from jax.experimental import pallas as pl


def kernel(x, edge_index, edge_attr, batch, Ws, bias):
    raise NotImplementedError("write your pallas kernel here")


# speedup vs baseline: 9.5701x; 9.5701x over previous
import functools

import jax
import jax.numpy as jnp
from jax import lax
from jax.experimental import pallas as pl
from jax.experimental.pallas import tpu as pltpu
from jax.experimental.pallas import tpu_sc as plsc

_N = 10000
_NP = 10240
_D = 128
_H = 64
_NE = 320000
_CB = 128
_CH = 160
_NE_T = _CB * _CH
_NE_PAD = 16 * _NE_T
_NT = 16
_NODES_T = _NP // _NT


def _sc_mesh():
    return plsc.VectorSubcoreMesh(core_axis_name="c", subcore_axis_name="s")


def _make_sc_propagate():
    f32 = jnp.float32
    out_type = (
        jax.ShapeDtypeStruct((2 * _NP, _H), f32),
        jax.ShapeDtypeStruct((2 * _NP, _H), f32),
        jax.ShapeDtypeStruct((2 * _NP, _H), f32),
        jax.ShapeDtypeStruct((2 * _NP, _H), f32),
        jax.ShapeDtypeStruct((_NP,), f32),
    )
    scratch = [
        pltpu.VMEM((_CH, _CB), jnp.int32),
        pltpu.VMEM((_CH, _CB), jnp.int32),
        pltpu.VMEM((_CB, _H), f32),
        pltpu.VMEM((_CB, _H), f32),
        pltpu.VMEM((_NP // _H, _H), f32),
        pltpu.VMEM((32, _H), f32),
        pltpu.VMEM((_NT, _NODES_T // _H, _H), f32),
        pltpu.VMEM((_NODES_T,), f32),
        pltpu.VMEM((_NODES_T,), f32),
        pltpu.VMEM((_NODES_T,), f32),
        pltpu.VMEM_SHARED((_NP, _H), f32),
        pltpu.SemaphoreType.DMA,
        pltpu.SemaphoreType.DMA,
    ]

    @functools.partial(pl.kernel, out_type=out_type, mesh=_sc_mesh(),
                       scratch_types=scratch,
                       compiler_params=pltpu.CompilerParams(
                           needs_layout_passes=False,
                           use_tc_tiling_on_sc=False))
    def sc_propagate(x2, row2, col3, p0, p1, p2, p3, s3,
                     rowbuf, colbuf, gbuf0, gbuf1, sbuf, zbuf,
                     sumb, disb, dis2b, s3b, accS,
                     gsem0, gsem1):
        c = lax.axis_index("c")
        s = lax.axis_index("s")
        base = s * _NODES_T
        zeros16 = jnp.zeros((16,), f32)

        eb = s * _CH
        pltpu.sync_copy(row2.at[pl.ds(c * (_NT * _CH) + eb, _CH)], rowbuf)
        pltpu.sync_copy(col3.at[pl.ds(eb, _CH)], colbuf)

        def _zb(m, _):
            for r in range(4):
                zbuf[m, pl.ds(r * 16, 16)] = zeros16
            return 0
        lax.fori_loop(0, 32, _zb, 0)

        def _zd(m, _):
            for r in range(4):
                sbuf[m, pl.ds(r * 16, 16)] = zeros16
            return 0
        lax.fori_loop(0, _NP // _H, _zd, 0)

        ones16 = jnp.ones((16,), f32)

        def _dg(g, _):
            for u in range(_CB // 16):
                cv = colbuf[g, pl.ds(u * 16, 16)]
                plsc.addupdate_scatter(sbuf, [cv >> 6, cv & 63], ones16)
            return 0
        lax.fori_loop(0, _CH, _dg, 0)

        pltpu.sync_copy(sbuf, accS.at[pl.ds(s * (_NP // _H), _NP // _H)])
        plsc.subcore_barrier()
        for t in range(_NT):
            pltpu.sync_copy(
                accS.at[pl.ds(t * (_NP // _H) + s * (_NODES_T // _H),
                              _NODES_T // _H)],
                sumb.at[t])
        plsc.subcore_barrier()

        def _nw(j, _):
            sl = pl.ds(j * 16, 16)
            r = j >> 2
            co = (j & 3) * 16
            d = sumb[0, r, pl.ds(co, 16)]
            for i in range(1, _NT):
                d = d + sumb[i, r, pl.ds(co, 16)]
            ii = lax.bitcast_convert_type(d, jnp.int32)
            y = lax.bitcast_convert_type(
                jnp.int32(0x5F3759DF) - (ii >> 1), f32)
            for _i in range(3):
                y = y * (1.5 - 0.5 * d * y * y)
            dis = jnp.where(d > 0.0, y, 0.0)
            disb[sl] = dis
            dis2b[sl] = dis * dis
            s3b[sl] = d * dis
            return 0
        lax.fori_loop(0, _NODES_T // 16, _nw, 0)

        @pl.when(c == 0)
        def _():
            pltpu.sync_copy(s3b, s3.at[pl.ds(base, _NODES_T)])

        _SC = _NP // _H

        def _pj(j, _):
            rb = base + j * _SC
            pltpu.sync_copy(x2.at[pl.ds(c * _NP + rb, _SC)], sbuf)

            def _pn(n, _n):
                nn = j * _SC + n
                i16 = jnp.zeros((16,), jnp.int32) + nn
                spl = plsc.load_gather(disb, [i16])
                for r in range(4):
                    sl = pl.ds(r * 16, 16)
                    sbuf[n, sl] = sbuf[n, sl] * spl
                return 0
            lax.fori_loop(0, _SC, _pn, 0)
            pltpu.sync_copy(sbuf, p0.at[pl.ds(c * _NP + rb, _SC)])
            return 0
        lax.fori_loop(0, _NODES_T // _SC, _pj, 0)
        plsc.subcore_barrier()

        bufs = [p0, p1, p2, p3]
        for k in (1, 2, 3):
            src = bufs[k - 1]
            dst = bufs[k]

            def _zr(m, _):
                pltpu.sync_copy(zbuf, accS.at[pl.ds(base + m * 32, 32)])
                return 0
            lax.fori_loop(0, _NODES_T // 32, _zr, 0)
            plsc.subcore_barrier()

            pltpu.async_copy(src.at[rowbuf.at[0]], gbuf0, gsem0)
            pltpu.async_copy(src.at[rowbuf.at[1]], gbuf1, gsem1)

            def _el(i, _):
                g0 = i * 2
                g1 = g0 + 1
                pltpu.make_async_copy(src.at[rowbuf.at[0]], gbuf0, gsem0).wait()
                pltpu.sync_copy(gbuf0, accS.at[colbuf.at[g0]], add=True)

                @pl.when(i < _CH // 2 - 1)
                def _():
                    pltpu.async_copy(src.at[rowbuf.at[g0 + 2]], gbuf0, gsem0)

                pltpu.make_async_copy(src.at[rowbuf.at[1]], gbuf1, gsem1).wait()
                pltpu.sync_copy(gbuf1, accS.at[colbuf.at[g1]], add=True)

                @pl.when(i < _CH // 2 - 1)
                def _():
                    pltpu.async_copy(src.at[rowbuf.at[g1 + 2]], gbuf1, gsem1)
                return 0
            lax.fori_loop(0, _CH // 2, _el, 0)
            plsc.subcore_barrier()

            def _dr(j, _):
                rb = base + j * _SC
                pltpu.sync_copy(accS.at[pl.ds(rb, _SC)], sbuf)

                def _dn(n, _n):
                    nn = j * _SC + n
                    i16 = jnp.zeros((16,), jnp.int32) + nn
                    sp2 = plsc.load_gather(dis2b, [i16])
                    for r in range(4):
                        sl = pl.ds(r * 16, 16)
                        sbuf[n, sl] = sbuf[n, sl] * sp2
                    return 0
                lax.fori_loop(0, _SC, _dn, 0)
                pltpu.sync_copy(sbuf, dst.at[pl.ds(c * _NP + rb, _SC)])
                return 0
            lax.fori_loop(0, _NODES_T // _SC, _dr, 0)
            plsc.subcore_barrier()

    return sc_propagate


_sc_propagate = _make_sc_propagate()


def _tc_body(x_ref, p10, p11, p20, p21, p30, p31, s3_ref, w_ref, b_ref, o_ref):
    acc = jnp.dot(x_ref[...], w_ref[0], preferred_element_type=jnp.float32)
    s3v = s3_ref[...]
    for k, (pa, pb) in enumerate(((p10, p11), (p20, p21), (p30, p31)), start=1):
        t = jnp.dot(pa[...], w_ref[k, :_H, :], preferred_element_type=jnp.float32)
        t = t + jnp.dot(pb[...], w_ref[k, _H:, :], preferred_element_type=jnp.float32)
        acc = acc + t * s3v
    o_ref[...] = jnp.maximum(acc + b_ref[...], 0.0)


_BLK = 640


def _tc_combine(xpad, p1, p2, p3, s3r, Ws, bias2):
    nblk = _NP // _BLK
    phalf = lambda off: pl.BlockSpec((_BLK, _H), lambda i, o=off: (i + o, 0))
    return pl.pallas_call(
        _tc_body,
        grid=(nblk,),
        in_specs=[
            pl.BlockSpec((_BLK, _D), lambda i: (i, 0)),
            phalf(0), phalf(nblk),
            phalf(0), phalf(nblk),
            phalf(0), phalf(nblk),
            pl.BlockSpec((_BLK, 1), lambda i: (i, 0)),
            pl.BlockSpec((4, _D, _D), lambda i: (0, 0, 0)),
            pl.BlockSpec((1, _D), lambda i: (0, 0)),
        ],
        out_specs=pl.BlockSpec((_BLK, _D), lambda i: (i, 0)),
        out_shape=jax.ShapeDtypeStruct((_NP, _D), jnp.float32),
    )(xpad, p1, p1, p2, p2, p3, p3, s3r, Ws, bias2)


def kernel(x, edge_index, edge_attr, batch, Ws, bias):
    f32 = jnp.float32
    row = edge_index[0].astype(jnp.int32)
    col = edge_index[1].astype(jnp.int32)
    padv = jnp.full((_NE_PAD - _NE,), _NP - 1, jnp.int32)
    rowp = jnp.concatenate([row, padv])
    colp = jnp.concatenate([col, padv])
    row2 = jnp.stack([rowp, rowp + _NP]).reshape(2 * _NT * _CH, _CB)
    col3 = colp.reshape(_NT * _CH, _CB)
    x = x.astype(f32)
    x2 = (jnp.zeros((2 * _NP, _H), f32)
          .at[:_N].set(x[:, :_H])
          .at[_NP:_NP + _N].set(x[:, _H:]))

    p0, p1, p2, p3, s3 = _sc_propagate(x2, row2, col3)

    xpad = jnp.zeros((_NP, _D), f32).at[:_N].set(x)
    out = _tc_combine(xpad, p1, p2, p3, s3.reshape(_NP, 1),
                      Ws.astype(f32), bias.astype(f32).reshape(1, _D))
    return out[:_N]
